# baseline (speedup 1.0000x reference)
import functools

import jax
import jax.numpy as jnp
from jax import lax
from jax.experimental import pallas as pl
from jax.experimental.pallas import tpu as pltpu
from jax.experimental.pallas import tpu_sc as plsc

NC = 2
NS = 16
C = 80
RPT = 624


def _sc_segment_sum(x, edge_attr, edge_index):
    N, D = x.shape
    E = edge_index.shape[1]
    NW = NC * NS
    assert E % (NW * C) == 0
    epw = E // NW
    iters = epw // C
    tail = N - RPT * NS
    assert 0 <= tail and tail % 8 == 0

    mesh = plsc.VectorSubcoreMesh(core_axis_name="c", subcore_axis_name="s")

    assert iters % 4 == 1

    ei_flat = edge_index.reshape(2 * E)

    @functools.partial(
        pl.kernel,
        mesh=mesh,
        out_type=jax.ShapeDtypeStruct((NC, N, D), jnp.float32),
        scratch_types=[
            pltpu.VMEM_SHARED((N, D), jnp.float32),
            pltpu.VMEM((2, C), jnp.int32),
            pltpu.VMEM((2, C), jnp.int32),
            pltpu.VMEM((2, C), jnp.int32),
            pltpu.VMEM((2, C), jnp.int32),
            pltpu.VMEM((C, D), jnp.float32),
            pltpu.VMEM((C, D), jnp.float32),
            pltpu.VMEM((C, D), jnp.float32),
            pltpu.VMEM((C, D), jnp.float32),
            pltpu.SemaphoreType.DMA,
            pltpu.SemaphoreType.DMA,
            pltpu.SemaphoreType.DMA,
            pltpu.SemaphoreType.DMA,
            pltpu.SemaphoreType.DMA,
            pltpu.SemaphoreType.DMA,
            pltpu.SemaphoreType.DMA,
            pltpu.SemaphoreType.DMA,
            pltpu.SemaphoreType.DMA,
            pltpu.SemaphoreType.DMA,
        ],
    )
    def k(x_hbm, ea_hbm, ei_hbm, out_hbm, h_sh, idx0, idx1, idx2, idx3,
          rows0, rows1, ea0, ea1, isem0, isem1, isem2, isem3,
          gsem0, gsem1, esem0, esem1, ssem0, ssem1):
        idx = (idx0, idx1, idx2, idx3)
        rows = (rows0, rows1)
        ea = (ea0, ea1)
        isem = (isem0, isem1, isem2, isem3)
        gsem = (gsem0, gsem1)
        esem = (esem0, esem1)
        ssem = (ssem0, ssem1)
        c = lax.axis_index("c")
        s = lax.axis_index("s")
        base = (c * NS + s) * epw
        row0 = s * RPT

        def fire_idx(i, q):
            e0 = base + i * C
            pltpu.async_copy(ei_hbm.at[pl.ds(e0, C)], idx[q].at[0], isem[q])
            pltpu.async_copy(ei_hbm.at[pl.ds(E + e0, C)], idx[q].at[1],
                             isem[q])

        def drain_idx(q):
            pltpu.make_async_copy(ei_hbm.at[pl.ds(0, C)], idx[q].at[0],
                                  isem[q]).wait()
            pltpu.make_async_copy(ei_hbm.at[pl.ds(0, C)], idx[q].at[1],
                                  isem[q]).wait()

        def fire_data(i, b, q):
            e0 = base + i * C
            pltpu.async_copy(x_hbm.at[idx[q].at[0]], rows[b], gsem[b])
            pltpu.async_copy(ea_hbm.at[pl.ds(e0, C)], ea[b], esem[b])

        def drain_data(b):
            pltpu.make_async_copy(ea_hbm.at[pl.ds(0, C)], rows[b],
                                  gsem[b]).wait()
            pltpu.make_async_copy(ea_hbm.at[pl.ds(0, C)], ea[b],
                                  esem[b]).wait()

        def fire_scatter(b, q):
            pltpu.async_copy(rows[b], h_sh.at[idx[q].at[1]], ssem[b],
                             add=True)
            pltpu.async_copy(ea[b], h_sh.at[idx[q].at[1]], ssem[b], add=True)

        def drain_scatter(b):
            pltpu.make_async_copy(ea_hbm.at[pl.ds(0, C)], rows[b],
                                  ssem[b]).wait()
            pltpu.make_async_copy(ea_hbm.at[pl.ds(0, C)], ea[b],
                                  ssem[b]).wait()

        fire_idx(0, 0)
        fire_idx(1, 1)

        def zfill(i, _):
            r = i // (D // 16)
            col = (i % (D // 16)) * 16
            rows1[r, pl.ds(col, 16)] = jnp.zeros((16,), jnp.float32)
            return 0
        lax.fori_loop(0, C * (D // 16), zfill, 0)

        drain_idx(0)
        fire_data(0, 0, 0)

        nzc = RPT // C
        def zcopy(j, _):
            pltpu.sync_copy(rows1, h_sh.at[pl.ds(row0 + j * C, C)])
            return 0
        lax.fori_loop(0, nzc, zcopy, 0)
        zrem = RPT - nzc * C
        if zrem:
            pltpu.sync_copy(rows1.at[pl.ds(0, zrem)],
                            h_sh.at[pl.ds(row0 + nzc * C, zrem)])
        if tail:
            @pl.when(s == NS - 1)
            def _():
                pltpu.sync_copy(rows1.at[pl.ds(0, tail)],
                                h_sh.at[pl.ds(NS * RPT, tail)])
        plsc.subcore_barrier()

        def quad(j, _):
            for u in range(4):
                b = u % 2
                i = 4 * j + u
                if u >= 2:
                    @pl.when(i + 2 < iters)
                    def _():
                        fire_idx(i + 2, (u + 2) % 4)
                else:
                    fire_idx(i + 2, (u + 2) % 4)
                drain_data(b)
                if u == 0:
                    @pl.when(j > 0)
                    def _():
                        drain_scatter(1 - b)
                else:
                    drain_scatter(1 - b)
                drain_idx((u + 1) % 4)
                fire_data(i + 1, 1 - b, (u + 1) % 4)
                fire_scatter(b, u)
            return 0
        lax.fori_loop(0, iters // 4, quad, 0)

        drain_scatter(1)
        drain_data(0)
        fire_scatter(0, 0)
        drain_scatter(0)
        plsc.subcore_barrier()

        pltpu.sync_copy(h_sh.at[pl.ds(row0, RPT)],
                        out_hbm.at[c, pl.ds(row0, RPT)])
        if tail:
            @pl.when(s == NS - 1)
            def _():
                pltpu.sync_copy(h_sh.at[pl.ds(NS * RPT, tail)],
                                out_hbm.at[c, pl.ds(NS * RPT, tail)])

    return k(x, edge_attr, ei_flat)


def _mlp_body(p_ref, w1_ref, b1_ref, w2_ref, b2_ref, out_ref):
    h = p_ref[0] + p_ref[1]
    h = jnp.dot(h, w1_ref[...], preferred_element_type=jnp.float32) + b1_ref[...]
    h = jnp.maximum(h, 0.0)
    out_ref[...] = (
        jnp.dot(h, w2_ref[...], preferred_element_type=jnp.float32) + b2_ref[...]
    )


def _mlp(partials, W1t, b1, W2t, b2):
    _, N, D = partials.shape
    D_out = W2t.shape[1]
    BN = 2000
    grid = (N // BN,)
    return pl.pallas_call(
        _mlp_body,
        grid=grid,
        in_specs=[
            pl.BlockSpec((NC, BN, D), lambda i: (0, i, 0)),
            pl.BlockSpec((D, W1t.shape[1]), lambda i: (0, 0)),
            pl.BlockSpec((1, W1t.shape[1]), lambda i: (0, 0)),
            pl.BlockSpec((W2t.shape[0], D_out), lambda i: (0, 0)),
            pl.BlockSpec((1, D_out), lambda i: (0, 0)),
        ],
        out_specs=pl.BlockSpec((BN, D_out), lambda i: (i, 0)),
        out_shape=jax.ShapeDtypeStruct((N, D_out), jnp.float32),
    )(partials, W1t, b1, W2t, b2)


def kernel(x, edge_attr, edge_index, W1, b1, W2, b2):
    partials = _sc_segment_sum(x, edge_attr, edge_index)
    return _mlp(partials, W1.T, b1.reshape(1, -1), W2.T, b2.reshape(1, -1))


# speedup vs baseline: 1.0077x; 1.0077x over previous
import functools

import jax
import jax.numpy as jnp
from jax import lax
from jax.experimental import pallas as pl
from jax.experimental.pallas import tpu as pltpu
from jax.experimental.pallas import tpu_sc as plsc

NC = 2
NS = 16
C = 80
RPT = 624


def _sc_segment_sum(x, edge_attr, edge_index):
    N, D = x.shape
    E = edge_index.shape[1]
    NW = NC * NS
    assert E % (NW * C) == 0
    epw = E // NW
    iters = epw // C
    tail = N - RPT * NS
    assert 0 <= tail and tail % 8 == 0

    mesh = plsc.VectorSubcoreMesh(core_axis_name="c", subcore_axis_name="s")

    assert iters % 4 == 1

    ei_flat = edge_index.reshape(2 * E)

    @functools.partial(
        pl.kernel,
        mesh=mesh,
        out_type=jax.ShapeDtypeStruct((NC, N, D), jnp.float32),
        scratch_types=[
            pltpu.VMEM_SHARED((N, D), jnp.float32),
            pltpu.VMEM((2, C), jnp.int32),
            pltpu.VMEM((2, C), jnp.int32),
            pltpu.VMEM((2, C), jnp.int32),
            pltpu.VMEM((2, C), jnp.int32),
            pltpu.VMEM((C, D), jnp.float32),
            pltpu.VMEM((C, D), jnp.float32),
            pltpu.VMEM((C, D), jnp.float32),
            pltpu.VMEM((C, D), jnp.float32),
            pltpu.SemaphoreType.DMA,
            pltpu.SemaphoreType.DMA,
            pltpu.SemaphoreType.DMA,
            pltpu.SemaphoreType.DMA,
            pltpu.SemaphoreType.DMA,
            pltpu.SemaphoreType.DMA,
            pltpu.SemaphoreType.DMA,
            pltpu.SemaphoreType.DMA,
            pltpu.SemaphoreType.DMA,
            pltpu.SemaphoreType.DMA,
        ],
    )
    def k(x_hbm, ea_hbm, ei_hbm, out_hbm, h_sh, idx0, idx1, idx2, idx3,
          rows0, rows1, ea0, ea1, isem0, isem1, isem2, isem3,
          gsem0, gsem1, esem0, esem1, ssem0, ssem1):
        idx = (idx0, idx1, idx2, idx3)
        rows = (rows0, rows1)
        ea = (ea0, ea1)
        isem = (isem0, isem1, isem2, isem3)
        gsem = (gsem0, gsem1)
        esem = (esem0, esem1)
        ssem = (ssem0, ssem1)
        c = lax.axis_index("c")
        s = lax.axis_index("s")
        base = (c * NS + s) * epw
        row0 = s * RPT

        def fire_idx(i, q):
            e0 = base + i * C
            pltpu.async_copy(ei_hbm.at[pl.ds(e0, C)], idx[q].at[0], isem[q])
            pltpu.async_copy(ei_hbm.at[pl.ds(E + e0, C)], idx[q].at[1],
                             isem[q])

        def drain_idx(q):
            pltpu.make_async_copy(ei_hbm.at[pl.ds(0, C)], idx[q].at[0],
                                  isem[q]).wait()
            pltpu.make_async_copy(ei_hbm.at[pl.ds(0, C)], idx[q].at[1],
                                  isem[q]).wait()

        def fire_data(i, b, q):
            e0 = base + i * C
            pltpu.async_copy(x_hbm.at[idx[q].at[0]], rows[b], gsem[b])
            pltpu.async_copy(ea_hbm.at[pl.ds(e0, C)], ea[b], esem[b])

        def drain_data(b):
            pltpu.make_async_copy(ea_hbm.at[pl.ds(0, C)], rows[b],
                                  gsem[b]).wait()
            pltpu.make_async_copy(ea_hbm.at[pl.ds(0, C)], ea[b],
                                  esem[b]).wait()

        def fire_scatter(b, q):
            pltpu.async_copy(rows[b], h_sh.at[idx[q].at[1]], ssem[b],
                             add=True)
            pltpu.async_copy(ea[b], h_sh.at[idx[q].at[1]], ssem[b], add=True)

        def drain_scatter(b):
            pltpu.make_async_copy(ea_hbm.at[pl.ds(0, C)], rows[b],
                                  ssem[b]).wait()
            pltpu.make_async_copy(ea_hbm.at[pl.ds(0, C)], ea[b],
                                  ssem[b]).wait()

        fire_idx(0, 0)
        fire_idx(1, 1)

        def zfill(i, _):
            r = i // (D // 16)
            col = (i % (D // 16)) * 16
            rows1[r, pl.ds(col, 16)] = jnp.zeros((16,), jnp.float32)
            return 0
        lax.fori_loop(0, C * (D // 16), zfill, 0)

        drain_idx(0)
        fire_data(0, 0, 0)

        nzc = RPT // C
        def zcopy(j, _):
            pltpu.sync_copy(rows1, h_sh.at[pl.ds(row0 + j * C, C)])
            return 0
        lax.fori_loop(0, nzc, zcopy, 0)
        zrem = RPT - nzc * C
        if zrem:
            pltpu.sync_copy(rows1.at[pl.ds(0, zrem)],
                            h_sh.at[pl.ds(row0 + nzc * C, zrem)])
        if tail:
            @pl.when(s == NS - 1)
            def _():
                pltpu.sync_copy(rows1.at[pl.ds(0, tail)],
                                h_sh.at[pl.ds(NS * RPT, tail)])
        plsc.subcore_barrier()

        def quad(j, _):
            for u in range(4):
                b = u % 2
                i = 4 * j + u
                if u == 0:
                    @pl.when(j > 0)
                    def _():
                        drain_scatter(1 - b)
                else:
                    drain_scatter(1 - b)
                if u >= 2:
                    @pl.when(i + 2 < iters)
                    def _():
                        fire_idx(i + 2, (u + 2) % 4)
                else:
                    fire_idx(i + 2, (u + 2) % 4)
                drain_data(b)
                drain_idx((u + 1) % 4)
                fire_data(i + 1, 1 - b, (u + 1) % 4)
                fire_scatter(b, u)
            return 0
        lax.fori_loop(0, iters // 4, quad, 0)

        drain_scatter(1)
        drain_data(0)
        fire_scatter(0, 0)
        drain_scatter(0)
        plsc.subcore_barrier()

        pltpu.sync_copy(h_sh.at[pl.ds(row0, RPT)],
                        out_hbm.at[c, pl.ds(row0, RPT)])
        if tail:
            @pl.when(s == NS - 1)
            def _():
                pltpu.sync_copy(h_sh.at[pl.ds(NS * RPT, tail)],
                                out_hbm.at[c, pl.ds(NS * RPT, tail)])

    return k(x, edge_attr, ei_flat)


def _mlp_body(p_ref, w1_ref, b1_ref, w2_ref, b2_ref, out_ref):
    h = p_ref[0] + p_ref[1]
    h = jnp.dot(h, w1_ref[...], preferred_element_type=jnp.float32) + b1_ref[...]
    h = jnp.maximum(h, 0.0)
    out_ref[...] = (
        jnp.dot(h, w2_ref[...], preferred_element_type=jnp.float32) + b2_ref[...]
    )


def _mlp(partials, W1t, b1, W2t, b2):
    _, N, D = partials.shape
    D_out = W2t.shape[1]
    BN = 2000
    grid = (N // BN,)
    return pl.pallas_call(
        _mlp_body,
        grid=grid,
        in_specs=[
            pl.BlockSpec((NC, BN, D), lambda i: (0, i, 0)),
            pl.BlockSpec((D, W1t.shape[1]), lambda i: (0, 0)),
            pl.BlockSpec((1, W1t.shape[1]), lambda i: (0, 0)),
            pl.BlockSpec((W2t.shape[0], D_out), lambda i: (0, 0)),
            pl.BlockSpec((1, D_out), lambda i: (0, 0)),
        ],
        out_specs=pl.BlockSpec((BN, D_out), lambda i: (i, 0)),
        out_shape=jax.ShapeDtypeStruct((N, D_out), jnp.float32),
    )(partials, W1t, b1, W2t, b2)


def kernel(x, edge_attr, edge_index, W1, b1, W2, b2):
    partials = _sc_segment_sum(x, edge_attr, edge_index)
    return _mlp(partials, W1.T, b1.reshape(1, -1), W2.T, b2.reshape(1, -1))
